# baseline (speedup 1.0000x reference)
import functools

import jax
import jax.numpy as jnp
from jax import lax
from jax.experimental import pallas as pl
from jax.experimental.pallas import tpu as pltpu
from jax.experimental.pallas import tpu_sc as plsc

D_EDGE = 16
N_BINS = 7
W_OUT = D_EDGE + N_BINS
LANES = 16
NC, NS = 2, 16
NW = NC * NS


@functools.lru_cache(maxsize=None)
def _build(E: int, C: int):
    per_w = E // NW
    n_chunks = per_w // C
    n_groups = C // LANES
    mesh = plsc.VectorSubcoreMesh(core_axis_name="c", subcore_axis_name="s")

    @functools.partial(
        pl.kernel,
        mesh=mesh,
        compiler_params=pltpu.CompilerParams(
            needs_layout_passes=False, use_tc_tiling_on_sc=False
        ),
        out_type=jax.ShapeDtypeStruct((W_OUT, E), jnp.float32),
        scratch_types=[
            pltpu.VMEM((C,), jnp.int32),
            pltpu.VMEM((C,), jnp.int32),
            pltpu.VMEM((N_BINS, C), jnp.float32),
            pltpu.SemaphoreType.DMA,
        ],
    )
    def k(s_hbm, r_hbm, ft_hbm, out_hbm, s_v, r_v, o7_v, fsem):
        wid = lax.axis_index("s") * NC + lax.axis_index("c")
        w_base = wid * per_w

        fcopy = pltpu.async_copy(
            ft_hbm.at[:, pl.ds(w_base, per_w)],
            out_hbm.at[pl.ds(0, D_EDGE), pl.ds(w_base, per_w)],
            fsem,
        )

        def chunk_body(ci, carry):
            base = w_base + ci * C
            pltpu.sync_copy(s_hbm.at[pl.ds(base, C)], s_v)
            pltpu.sync_copy(r_hbm.at[pl.ds(base, C)], r_v)

            def group_body(g, gcarry):
                gb = g * LANES
                s = s_v[pl.ds(gb, LANES)]
                r = r_v[pl.ds(gb, LANES)]
                sep = s - r + 1
                idx = jnp.clip(sep + 5, 0, 6) + jnp.clip(sep + 10, 0, 1)
                cls = 6 - idx
                for c in range(N_BINS):
                    vals = jnp.where(cls == c, 1.0, 0.0).astype(jnp.float32)
                    o7_v[c, pl.ds(gb, LANES)] = vals
                return gcarry

            lax.fori_loop(0, n_groups, group_body, 0)
            pltpu.sync_copy(
                o7_v, out_hbm.at[pl.ds(D_EDGE, N_BINS), pl.ds(base, C)]
            )
            return carry

        lax.fori_loop(0, n_chunks, chunk_body, 0)
        fcopy.wait()

    return k


def kernel(senders, receivers, edge_features):
    E = senders.shape[0]
    C = 2000
    assert E % (NW * C) == 0
    k = _build(E, C)
    out_t = k(senders, receivers, edge_features.T)
    return out_t.T


# speedup vs baseline: 2.9582x; 2.9582x over previous
import functools

import jax
import jax.numpy as jnp
from jax import lax
from jax.experimental import pallas as pl
from jax.experimental.pallas import tpu as pltpu
from jax.experimental.pallas import tpu_sc as plsc

D_EDGE = 16
N_BINS = 7
W_OUT = D_EDGE + N_BINS
LANES = 16
NC, NS = 2, 16
NW = NC * NS
RB_IN = D_EDGE // 8
RB_OUT = W_OUT // 8 + 1


@functools.lru_cache(maxsize=None)
def _build(E: int, CB: int):
    n_cb = E // 128
    n_chunks = (n_cb + CB - 1) // CB
    assert n_cb % CB == 0
    per_w = (n_chunks + NW - 1) // NW
    C = CB * 128
    n_groups = C // LANES
    mesh = plsc.VectorSubcoreMesh(core_axis_name="c", subcore_axis_name="s")

    @functools.partial(
        pl.kernel,
        mesh=mesh,
        compiler_params=pltpu.CompilerParams(
            needs_layout_passes=False, use_tc_tiling_on_sc=False
        ),
        out_type=jax.ShapeDtypeStruct((RB_OUT, n_cb, 8, 128), jnp.float32),
        scratch_types=[
            pltpu.VMEM((C,), jnp.int32),
            pltpu.VMEM((C,), jnp.int32),
            pltpu.VMEM((CB, 8, 128), jnp.float32),
            pltpu.SemaphoreType.DMA,
        ],
    )
    def k(s_hbm, r_hbm, x4_hbm, o4_hbm, s_v, r_v, ob_v, fsem):
        wid = lax.axis_index("s") * NC + lax.axis_index("c")

        def chunk_body(j, carry):
            ck = wid + j * NW

            @pl.when(ck < n_chunks)
            def _():
                cb0 = ck * CB
                e0 = cb0 * 128
                fcopy = pltpu.async_copy(
                    x4_hbm.at[:, pl.ds(cb0, CB), :, :],
                    o4_hbm.at[pl.ds(0, RB_IN), pl.ds(cb0, CB), :, :],
                    fsem,
                )
                pltpu.sync_copy(s_hbm.at[pl.ds(e0, C)], s_v)
                pltpu.sync_copy(r_hbm.at[pl.ds(e0, C)], r_v)

                def group_body(g, gcarry):
                    gb = g * LANES
                    s = s_v[pl.ds(gb, LANES)]
                    r = r_v[pl.ds(gb, LANES)]
                    sep = s - r + 1
                    idx = jnp.clip(sep + 5, 0, 6) + jnp.clip(sep + 10, 0, 1)
                    cls = 6 - idx
                    b = g >> 3
                    off = (g & 7) * LANES
                    for c in range(N_BINS):
                        vals = jnp.where(cls == c, 1.0, 0.0).astype(jnp.float32)
                        ob_v[b, c, pl.ds(off, LANES)] = vals
                    return gcarry

                lax.fori_loop(0, n_groups, group_body, 0)
                pltpu.sync_copy(ob_v, o4_hbm.at[2, pl.ds(cb0, CB), :, :])
                fcopy.wait()

            return carry

        lax.fori_loop(0, per_w, chunk_body, 0)

    return k


def kernel(senders, receivers, edge_features):
    E = senders.shape[0]
    assert E % 128 == 0
    n_cb = E // 128
    CB = 50
    k = _build(E, CB)
    x4 = edge_features.T.reshape(RB_IN, 8, n_cb, 128).transpose(0, 2, 1, 3)
    o4 = k(senders, receivers, x4)
    out_t = o4.transpose(0, 2, 1, 3).reshape(RB_OUT * 8, E)
    return out_t[:W_OUT].T


# speedup vs baseline: 39.0803x; 13.2109x over previous
import functools

import jax
import jax.numpy as jnp
from jax import lax
from jax.experimental import pallas as pl
from jax.experimental.pallas import tpu as pltpu
from jax.experimental.pallas import tpu_sc as plsc

D_EDGE = 16
N_BINS = 7
W_OUT = D_EDGE + N_BINS
LANES = 16
NC, NS = 2, 16
NW = NC * NS
RB_IN = D_EDGE // 8
RB_OUT = W_OUT // 8 + 1


@functools.lru_cache(maxsize=None)
def _build(E: int, CB: int):
    n_cb = E // 128
    n_chunks = (n_cb + CB - 1) // CB
    assert n_cb % CB == 0
    per_w = (n_chunks + NW - 1) // NW
    C = CB * 128
    n_groups = C // LANES
    mesh = plsc.VectorSubcoreMesh(core_axis_name="c", subcore_axis_name="s")

    @functools.partial(
        pl.kernel,
        mesh=mesh,
        compiler_params=pltpu.CompilerParams(
            needs_layout_passes=False, use_tc_tiling_on_sc=False
        ),
        out_type=jax.ShapeDtypeStruct((RB_OUT, n_cb, 8, 128), jnp.float32),
        scratch_types=[
            pltpu.VMEM((C,), jnp.int32),
            pltpu.VMEM((C,), jnp.int32),
            pltpu.VMEM((CB, 8, 128), jnp.float32),
            pltpu.VMEM((RB_IN, CB, 8, 128), jnp.float32),
            pltpu.SemaphoreType.DMA,
        ],
    )
    def k(s_hbm, r_hbm, x4_hbm, o4_hbm, s_v, r_v, ob_v, fb_v, fsem):
        wid = lax.axis_index("s") * NC + lax.axis_index("c")

        def chunk_body(j, carry):
            ck = wid + j * NW

            @pl.when(ck < n_chunks)
            def _():
                cb0 = ck * CB
                e0 = cb0 * 128
                fin = pltpu.async_copy(
                    x4_hbm.at[:, pl.ds(cb0, CB), :, :], fb_v, fsem
                )
                pltpu.sync_copy(s_hbm.at[pl.ds(e0, C)], s_v)
                pltpu.sync_copy(r_hbm.at[pl.ds(e0, C)], r_v)

                def group_body(g, gcarry):
                    gb = g * LANES
                    s = s_v[pl.ds(gb, LANES)]
                    r = r_v[pl.ds(gb, LANES)]
                    sep = s - r + 1
                    idx = jnp.clip(sep + 5, 0, 6) + jnp.clip(sep + 10, 0, 1)
                    cls = 6 - idx
                    b = g >> 3
                    off = (g & 7) * LANES
                    for c in range(N_BINS):
                        vals = jnp.where(cls == c, 1.0, 0.0).astype(jnp.float32)
                        ob_v[b, c, pl.ds(off, LANES)] = vals
                    return gcarry

                lax.fori_loop(0, n_groups, group_body, 0)
                pltpu.sync_copy(ob_v, o4_hbm.at[2, pl.ds(cb0, CB), :, :])
                fin.wait()
                pltpu.sync_copy(
                    fb_v, o4_hbm.at[pl.ds(0, RB_IN), pl.ds(cb0, CB), :, :]
                )

            return carry

        lax.fori_loop(0, per_w, chunk_body, 0)

    return k


def kernel(senders, receivers, edge_features):
    E = senders.shape[0]
    assert E % 128 == 0
    n_cb = E // 128
    CB = 25
    k = _build(E, CB)
    x4 = edge_features.T.reshape(RB_IN, 8, n_cb, 128).transpose(0, 2, 1, 3)
    o4 = k(senders, receivers, x4)
    out_t = o4.transpose(0, 2, 1, 3).reshape(RB_OUT * 8, E)
    return out_t[:W_OUT].T


# speedup vs baseline: 63.9522x; 1.6364x over previous
import functools

import jax
import jax.numpy as jnp
from jax import lax
from jax.experimental import pallas as pl
from jax.experimental.pallas import tpu as pltpu
from jax.experimental.pallas import tpu_sc as plsc

D_EDGE = 16
N_BINS = 7
W_OUT = D_EDGE + N_BINS
LANES = 16
NC, NS = 2, 16
NW = NC * NS


@functools.lru_cache(maxsize=None)
def _build(E: int, C: int):
    n_chunks = E // C
    n_groups = C // LANES
    per_w = (n_chunks + NW - 1) // NW
    mesh = plsc.VectorSubcoreMesh(core_axis_name="c", subcore_axis_name="s")

    @functools.partial(
        pl.kernel,
        mesh=mesh,
        compiler_params=pltpu.CompilerParams(
            needs_layout_passes=False, use_tc_tiling_on_sc=True
        ),
        out_type=jax.ShapeDtypeStruct((W_OUT, E), jnp.float32),
        scratch_types=[
            pltpu.VMEM((C,), jnp.int32),
            pltpu.VMEM((C,), jnp.int32),
            pltpu.VMEM((N_BINS, C), jnp.float32),
            pltpu.VMEM((D_EDGE, C), jnp.float32),
            pltpu.SemaphoreType.DMA,
        ],
    )
    def k(s_hbm, r_hbm, x_hbm, o_hbm, s_v, r_v, ob_v, fb_v, fsem):
        wid = lax.axis_index("s") * NC + lax.axis_index("c")

        def chunk_body(j, carry):
            ck = wid + j * NW

            @pl.when(ck < n_chunks)
            def _():
                e0 = ck * C
                fin = pltpu.async_copy(
                    x_hbm.at[:, pl.ds(e0, C)], fb_v, fsem
                )
                pltpu.sync_copy(s_hbm.at[pl.ds(e0, C)], s_v)
                pltpu.sync_copy(r_hbm.at[pl.ds(e0, C)], r_v)

                def group_body(g, gcarry):
                    gb = g * LANES
                    s = s_v[pl.ds(gb, LANES)]
                    r = r_v[pl.ds(gb, LANES)]
                    sep = s - r + 1
                    idx = jnp.clip(sep + 5, 0, 6) + jnp.clip(sep + 10, 0, 1)
                    cls = 6 - idx
                    for c in range(N_BINS):
                        vals = jnp.where(cls == c, 1.0, 0.0).astype(jnp.float32)
                        ob_v[c, pl.ds(gb, LANES)] = vals
                    return gcarry

                lax.fori_loop(0, n_groups, group_body, 0)
                pltpu.sync_copy(
                    ob_v, o_hbm.at[pl.ds(D_EDGE, N_BINS), pl.ds(e0, C)]
                )
                fin.wait()
                pltpu.sync_copy(
                    fb_v, o_hbm.at[pl.ds(0, D_EDGE), pl.ds(e0, C)]
                )

            return carry

        lax.fori_loop(0, per_w, chunk_body, 0)

    return k


def kernel(senders, receivers, edge_features):
    E = senders.shape[0]
    C = 3200
    assert E % C == 0
    k = _build(E, C)
    out_t = k(senders, receivers, edge_features.T)
    return out_t.T
